# Initial kernel scaffold; baseline (speedup 1.0000x reference)
import jax
import jax.numpy as jnp
from jax.experimental import pallas as pl


def kernel(predictions, targets, x_input, w1, b1, w2, b2):
    raise NotImplementedError("write your pallas kernel here")


# speedup vs baseline: 58.5833x; 58.5833x over previous
import functools

import jax
import jax.numpy as jnp
from jax import lax
from jax.experimental import pallas as pl
from jax.experimental.pallas import tpu as pltpu
from jax.experimental.pallas import tpu_sc as plsc

_N = 8192
_L = 16
_NC, _NS = 2, 16
_CHUNK = _N // _NS
_ROWS = _N // (_NC * _NS)
_INF = 3.0e38

_BASE_WEIGHT = 0.01
_ADAPT = 0.1


_GDN = lax.GatherDimensionNumbers(
    offset_dims=(), collapsed_slice_dims=(0,), start_index_map=(0,))


def _lane_perm(v, idx16):
    return lax.gather(v, jnp.reshape(idx16, (_L, 1)), _GDN, (1,),
                      mode=lax.GatherScatterMode.PROMISE_IN_BOUNDS)


def _lanes():
    return lax.iota(jnp.int32, _L)


def _flip16(v):
    return _lane_perm(v, (_L - 1) - _lanes())


def _ce(v, s, keepmin):
    p = _lane_perm(v, _lanes() ^ s)
    return jnp.where(keepmin, jnp.minimum(v, p), jnp.maximum(v, p))


def _sort16(v):
    lanes = _lanes()
    for k in (1, 2, 3, 4):
        for ls in range(k - 1, -1, -1):
            s = 1 << ls
            keepmin = ((lanes & s) == 0) ^ (((lanes >> k) & 1) == 1)
            v = _ce(v, s, keepmin)
    return v


def _clean16(v):
    lanes = _lanes()
    for s in (8, 4, 2, 1):
        v = _ce(v, s, (lanes & s) == 0)
    return v


def _bitonic_merge(ref, base, m):
    nb = m // _L

    def tri(i, c):
        a = ref[pl.ds(base + i * _L, _L)]
        b = ref[pl.ds(base + (2 * nb - 1 - i) * _L, _L)]
        br = _flip16(b)
        ref[pl.ds(base + i * _L, _L)] = jnp.minimum(a, br)
        ref[pl.ds(base + (2 * nb - 1 - i) * _L, _L)] = _flip16(
            jnp.maximum(a, br))
        return c

    lax.fori_loop(0, nb, tri, 0)

    s16 = m // (2 * _L)
    while s16 >= 1:
        def cle(i, c, s16=s16):
            blk = i // s16
            off = i - blk * s16
            j1 = blk * (2 * s16) + off
            a = ref[pl.ds(base + j1 * _L, _L)]
            b = ref[pl.ds(base + (j1 + s16) * _L, _L)]
            ref[pl.ds(base + j1 * _L, _L)] = jnp.minimum(a, b)
            ref[pl.ds(base + (j1 + s16) * _L, _L)] = jnp.maximum(a, b)
            return c

        lax.fori_loop(0, nb, cle, 0)
        s16 //= 2

    def vs(v, c):
        ref[pl.ds(base + v * _L, _L)] = _clean16(ref[pl.ds(base + v * _L, _L)])
        return c

    lax.fori_loop(0, 2 * nb, vs, 0)


def _knn_body(x_hbm, s_hbm, xs_spmem, work, out_vmem):
    cid = lax.axis_index("c")
    sid = lax.axis_index("s")

    pltpu.sync_copy(x_hbm.at[pl.ds(sid * _CHUNK, _CHUNK)],
                    work.at[pl.ds(0, _CHUNK)])

    def vs(v, c):
        work[pl.ds(v * _L, _L)] = _sort16(work[pl.ds(v * _L, _L)])
        return c

    lax.fori_loop(0, _CHUNK // _L, vs, 0)
    m = _L
    while m < _CHUNK:
        def task(t, c, m=m):
            _bitonic_merge(work, t * 2 * m, m)
            return c

        lax.fori_loop(0, _CHUNK // (2 * m), task, 0)
        m *= 2

    pltpu.sync_copy(work.at[pl.ds(0, _CHUNK)],
                    xs_spmem.at[pl.ds(sid * _CHUNK, _CHUNK)])
    plsc.subcore_barrier()

    def chunk_pair_phase(e1, e2, is_tri):
        e1 = pl.multiple_of(e1, _L)
        e2 = pl.multiple_of(e2, _L)
        pltpu.sync_copy(xs_spmem.at[pl.ds(e1, 256)], work.at[pl.ds(0, 256)])
        pltpu.sync_copy(xs_spmem.at[pl.ds(e2, 256)], work.at[pl.ds(256, 256)])
        for k in range(16):
            a = work[pl.ds(k * _L, _L)]
            if is_tri:
                b = _flip16(work[pl.ds((16 + (15 - k)) * _L, _L)])
                work[pl.ds(k * _L, _L)] = jnp.minimum(a, b)
                work[pl.ds((16 + (15 - k)) * _L, _L)] = _flip16(
                    jnp.maximum(a, b))
            else:
                b = work[pl.ds((16 + k) * _L, _L)]
                work[pl.ds(k * _L, _L)] = jnp.minimum(a, b)
                work[pl.ds((16 + k) * _L, _L)] = jnp.maximum(a, b)
        pltpu.sync_copy(work.at[pl.ds(0, 256)], xs_spmem.at[pl.ds(e1, 256)])
        pltpu.sync_copy(work.at[pl.ds(256, 256)], xs_spmem.at[pl.ds(e2, 256)])
        plsc.subcore_barrier()

    gp0 = sid * 16

    @pl.when(sid < 8)
    def _():
        eb = sid * 1024
        pltpu.sync_copy(xs_spmem.at[pl.ds(eb, 1024)], work.at[pl.ds(0, 1024)])
        _bitonic_merge(work, 0, 512)
        pltpu.sync_copy(work.at[pl.ds(0, 1024)], xs_spmem.at[pl.ds(eb, 1024)])

    plsc.subcore_barrier()

    def local1024_phase():
        @pl.when(sid < 8)
        def _():
            eb = sid * 1024
            pltpu.sync_copy(xs_spmem.at[pl.ds(eb, 1024)],
                            work.at[pl.ds(0, 1024)])

            def lpass(p, c):
                s16v = 32 >> p

                def pair(i, c2):
                    blk2 = i >> (5 - p)
                    off2 = i & (s16v - 1)
                    j1 = blk2 * 2 * s16v + off2
                    a = work[pl.ds(j1 * _L, _L)]
                    b = work[pl.ds((j1 + s16v) * _L, _L)]
                    work[pl.ds(j1 * _L, _L)] = jnp.minimum(a, b)
                    work[pl.ds((j1 + s16v) * _L, _L)] = jnp.maximum(a, b)
                    return c2

                lax.fori_loop(0, 32, pair, 0)
                return c

            lax.fori_loop(0, 6, lpass, 0)

            def cl(v, c):
                work[pl.ds(v * _L, _L)] = _clean16(work[pl.ds(v * _L, _L)])
                return c

            lax.fori_loop(0, 64, cl, 0)
            pltpu.sync_copy(work.at[pl.ds(0, 1024)],
                            xs_spmem.at[pl.ds(eb, 1024)])

        plsc.subcore_barrier()

    for m in (1024, 2048, 4096):
        nbv = m // _L
        t = gp0 // nbv
        i0 = gp0 - t * nbv
        chunk_pair_phase((t * 2 * nbv + i0) * _L,
                         (t * 2 * nbv + 2 * nbv - 16 - i0) * _L, True)
        s16 = nbv // 2
        while s16 >= 64:
            blk = i0 // s16
            off = i0 - blk * s16
            j1 = t * 2 * nbv + blk * 2 * s16 + off
            chunk_pair_phase(j1 * _L, (j1 + s16) * _L, False)
            s16 //= 2
        local1024_phase()

    grow = cid * (_NS * _ROWS) + sid * _ROWS
    cbase = pl.multiple_of(
        jnp.minimum(jnp.maximum(grow - _L, 0), _N - 288), _L)
    loff = pl.multiple_of(grow - cbase + _L, _L)
    pltpu.sync_copy(xs_spmem.at[pl.ds(cbase, 288)], work.at[pl.ds(_L, 288)])

    inf = jnp.full((_L,), _INF, jnp.float32)
    lanes = _lanes()
    idx_m1 = (lanes - 1) & (_L - 1)
    idx_m2 = (lanes - 2) & (_L - 1)
    idx_p1 = (lanes + 1) & (_L - 1)
    idx_p2 = (lanes + 2) & (_L - 1)
    for g in range(_ROWS // _L):
        li = loff + g * _L
        v0 = work[pl.ds(li, _L)]
        vprev = work[pl.ds(li - _L, _L)]
        vnext = work[pl.ds(li + _L, _L)]
        vm1 = jnp.where(lanes >= 1, _lane_perm(v0, idx_m1),
                        _lane_perm(vprev, idx_m1))
        vm2 = jnp.where(lanes >= 2, _lane_perm(v0, idx_m2),
                        _lane_perm(vprev, idx_m2))
        vp1 = jnp.where(lanes <= _L - 2, _lane_perm(v0, idx_p1),
                        _lane_perm(vnext, idx_p1))
        vp2 = jnp.where(lanes <= _L - 3, _lane_perm(v0, idx_p2),
                        _lane_perm(vnext, idx_p2))
        row = grow + g * _L + lax.iota(jnp.int32, _L)
        c1 = jnp.where(row >= 1, v0 - vm1, inf)
        c3 = jnp.where(row >= 2, v0 - vm2, inf)
        c2 = jnp.where(row <= _N - 2, vp1 - v0, inf)
        c4 = jnp.where(row <= _N - 3, vp2 - v0, inf)
        first = jnp.minimum(c1, c2)
        second = jnp.where(c1 <= c2, jnp.minimum(c2, c3),
                           jnp.minimum(c1, c4))
        out_vmem[pl.ds(g * _L, _L)] = first + second + jnp.float32(3e-8)

    pltpu.sync_copy(out_vmem, s_hbm.at[pl.ds(grow, _ROWS)])


@functools.cache
def _knn3_built():
    return functools.partial(
        pl.kernel,
        out_type=jax.ShapeDtypeStruct((_N,), jnp.float32),
        mesh=plsc.VectorSubcoreMesh(core_axis_name="c", subcore_axis_name="s",
                                    num_cores=_NC, num_subcores=_NS),
        scratch_types=[
            pltpu.VMEM_SHARED((_N,), jnp.float32),
            pltpu.VMEM((_N,), jnp.float32),
            pltpu.VMEM((_ROWS,), jnp.float32),
        ],
    )(_knn_body)


def _combine_body(params, x_ref, tg_ref, s_ref, total_ref, mse_ref, pen_ref):
    w1 = params[0]
    b1 = params[1]
    w2 = params[2]
    b2 = params[3]
    x = x_ref[...]
    t = jnp.tanh(w1 * x + b1)
    preds = w2 * t + b2
    r = preds - tg_ref[...]
    mse = jnp.sum(r * r) * jnp.float32(1.0 / _N)
    d2 = (-2.0 * w1 * w1 * w2) * t * (1.0 - t * t)
    msd2 = jnp.sum(d2 * d2) * jnp.float32(1.0 / _N)
    raw = 1.0 / (s_ref[...] * jnp.float32(1.0 / 3.0) + 1e-8)
    mean_density = jnp.sum(raw) * jnp.float32(1.0 / _N) / (jnp.max(raw) + 1e-8)
    pen = jnp.float32(_BASE_WEIGHT) * (1.0 + jnp.float32(_ADAPT) * mean_density) * msd2
    mse_ref[0, 0] = mse
    pen_ref[0, 0] = pen
    total_ref[0, 0] = mse + pen


_combine = pl.pallas_call(
    _combine_body,
    in_specs=[
        pl.BlockSpec(memory_space=pltpu.SMEM),
        pl.BlockSpec(memory_space=pltpu.VMEM),
        pl.BlockSpec(memory_space=pltpu.VMEM),
        pl.BlockSpec(memory_space=pltpu.VMEM),
    ],
    out_specs=[
        pl.BlockSpec(memory_space=pltpu.SMEM),
        pl.BlockSpec(memory_space=pltpu.SMEM),
        pl.BlockSpec(memory_space=pltpu.SMEM),
    ],
    out_shape=[
        jax.ShapeDtypeStruct((1, 1), jnp.float32),
        jax.ShapeDtypeStruct((1, 1), jnp.float32),
        jax.ShapeDtypeStruct((1, 1), jnp.float32),
    ],
)


def kernel(predictions, targets, x_input, w1, b1, w2, b2):
    x = x_input.astype(jnp.float32)
    s = _knn3_built()(x)
    params = jnp.stack([jnp.float32(w1), jnp.float32(b1),
                        jnp.float32(w2), jnp.float32(b2)])
    total, mse, pen = _combine(
        params,
        x.reshape(_N // 128, 128),
        targets.astype(jnp.float32).reshape(_N // 128, 128),
        s.reshape(_N // 128, 128),
    )
    return total[0, 0], mse[0, 0], pen[0, 0]


# speedup vs baseline: 67.4391x; 1.1512x over previous
import functools

import jax
import jax.numpy as jnp
from jax import lax
from jax.experimental import pallas as pl
from jax.experimental.pallas import tpu as pltpu
from jax.experimental.pallas import tpu_sc as plsc

_N = 8192
_L = 16
_NC, _NS = 2, 16
_CHUNK = _N // _NS
_ROWS = _N // (_NC * _NS)
_INF = 3.0e38

_BASE_WEIGHT = 0.01
_ADAPT = 0.1


_GDN = lax.GatherDimensionNumbers(
    offset_dims=(), collapsed_slice_dims=(0,), start_index_map=(0,))


def _lane_perm(v, idx16):
    return lax.gather(v, jnp.reshape(idx16, (_L, 1)), _GDN, (1,),
                      mode=lax.GatherScatterMode.PROMISE_IN_BOUNDS)


def _lanes():
    return lax.iota(jnp.int32, _L)


def _flip16(v):
    return _lane_perm(v, (_L - 1) - _lanes())


def _ce(v, s, keepmin):
    p = _lane_perm(v, _lanes() ^ s)
    return jnp.where(keepmin, jnp.minimum(v, p), jnp.maximum(v, p))


def _sort16(v):
    lanes = _lanes()
    for k in (1, 2, 3, 4):
        for ls in range(k - 1, -1, -1):
            s = 1 << ls
            keepmin = ((lanes & s) == 0) ^ (((lanes >> k) & 1) == 1)
            v = _ce(v, s, keepmin)
    return v


def _clean16(v):
    lanes = _lanes()
    for s in (8, 4, 2, 1):
        v = _ce(v, s, (lanes & s) == 0)
    return v


def _bitonic_merge(ref, base, m):
    nb = m // _L

    def tri(i, c):
        a = ref[pl.ds(base + i * _L, _L)]
        b = ref[pl.ds(base + (2 * nb - 1 - i) * _L, _L)]
        br = _flip16(b)
        ref[pl.ds(base + i * _L, _L)] = jnp.minimum(a, br)
        ref[pl.ds(base + (2 * nb - 1 - i) * _L, _L)] = _flip16(
            jnp.maximum(a, br))
        return c

    lax.fori_loop(0, nb, tri, 0, unroll=min(4, nb))

    s16 = m // (2 * _L)
    while s16 >= 1:
        def cle(i, c, s16=s16):
            blk = i // s16
            off = i - blk * s16
            j1 = blk * (2 * s16) + off
            a = ref[pl.ds(base + j1 * _L, _L)]
            b = ref[pl.ds(base + (j1 + s16) * _L, _L)]
            ref[pl.ds(base + j1 * _L, _L)] = jnp.minimum(a, b)
            ref[pl.ds(base + (j1 + s16) * _L, _L)] = jnp.maximum(a, b)
            return c

        lax.fori_loop(0, nb, cle, 0, unroll=min(4, nb))
        s16 //= 2

    def vs(v, c):
        ref[pl.ds(base + v * _L, _L)] = _clean16(ref[pl.ds(base + v * _L, _L)])
        return c

    lax.fori_loop(0, 2 * nb, vs, 0, unroll=min(4, 2 * nb))


def _knn_body(x_hbm, s_hbm, xs_spmem, work, out_vmem):
    cid = lax.axis_index("c")
    sid = lax.axis_index("s")

    pltpu.sync_copy(x_hbm.at[pl.ds(sid * _CHUNK, _CHUNK)],
                    work.at[pl.ds(0, _CHUNK)])

    def vs(v, c):
        work[pl.ds(v * _L, _L)] = _sort16(work[pl.ds(v * _L, _L)])
        return c

    lax.fori_loop(0, _CHUNK // _L, vs, 0, unroll=4)
    m = _L
    while m < _CHUNK:
        def task(t, c, m=m):
            _bitonic_merge(work, t * 2 * m, m)
            return c

        lax.fori_loop(0, _CHUNK // (2 * m), task, 0)
        m *= 2

    pltpu.sync_copy(work.at[pl.ds(0, _CHUNK)],
                    xs_spmem.at[pl.ds(sid * _CHUNK, _CHUNK)])
    plsc.subcore_barrier()

    def chunk_pair_phase(e1, e2, is_tri):
        e1 = pl.multiple_of(e1, _L)
        e2 = pl.multiple_of(e2, _L)
        pltpu.sync_copy(xs_spmem.at[pl.ds(e1, 256)], work.at[pl.ds(0, 256)])
        pltpu.sync_copy(xs_spmem.at[pl.ds(e2, 256)], work.at[pl.ds(256, 256)])
        for k in range(16):
            a = work[pl.ds(k * _L, _L)]
            if is_tri:
                b = _flip16(work[pl.ds((16 + (15 - k)) * _L, _L)])
                work[pl.ds(k * _L, _L)] = jnp.minimum(a, b)
                work[pl.ds((16 + (15 - k)) * _L, _L)] = _flip16(
                    jnp.maximum(a, b))
            else:
                b = work[pl.ds((16 + k) * _L, _L)]
                work[pl.ds(k * _L, _L)] = jnp.minimum(a, b)
                work[pl.ds((16 + k) * _L, _L)] = jnp.maximum(a, b)
        pltpu.sync_copy(work.at[pl.ds(0, 256)], xs_spmem.at[pl.ds(e1, 256)])
        pltpu.sync_copy(work.at[pl.ds(256, 256)], xs_spmem.at[pl.ds(e2, 256)])
        plsc.subcore_barrier()

    gp0 = sid * 16
    m = _CHUNK
    while m < _N:
        nbv = m // _L
        t = gp0 // nbv
        i0 = gp0 - t * nbv
        chunk_pair_phase((t * 2 * nbv + i0) * _L,
                         (t * 2 * nbv + 2 * nbv - 16 - i0) * _L, True)
        s16 = nbv // 2
        while s16 >= 32:
            blk = i0 // s16
            off = i0 - blk * s16
            j1 = t * 2 * nbv + blk * 2 * s16 + off
            chunk_pair_phase(j1 * _L, (j1 + s16) * _L, False)
            s16 //= 2
        ebase = sid * 512
        pltpu.sync_copy(xs_spmem.at[pl.ds(ebase, 512)],
                        work.at[pl.ds(0, 512)])

        def lpass(p, c):
            s16v = 16 >> p

            def pair(i, c2):
                blk2 = i >> (4 - p)
                off2 = i & (s16v - 1)
                j1 = blk2 * 2 * s16v + off2
                a = work[pl.ds(j1 * _L, _L)]
                b = work[pl.ds((j1 + s16v) * _L, _L)]
                work[pl.ds(j1 * _L, _L)] = jnp.minimum(a, b)
                work[pl.ds((j1 + s16v) * _L, _L)] = jnp.maximum(a, b)
                return c2

            lax.fori_loop(0, 16, pair, 0, unroll=4)
            return c

        lax.fori_loop(0, 5, lpass, 0)

        def cl(v, c):
            work[pl.ds(v * _L, _L)] = _clean16(work[pl.ds(v * _L, _L)])
            return c

        lax.fori_loop(0, 32, cl, 0, unroll=4)
        pltpu.sync_copy(work.at[pl.ds(0, 512)],
                        xs_spmem.at[pl.ds(ebase, 512)])
        plsc.subcore_barrier()
        m *= 2

    grow = cid * (_NS * _ROWS) + sid * _ROWS
    cbase = pl.multiple_of(
        jnp.minimum(jnp.maximum(grow - _L, 0), _N - 288), _L)
    loff = pl.multiple_of(grow - cbase + _L, _L)
    pltpu.sync_copy(xs_spmem.at[pl.ds(cbase, 288)], work.at[pl.ds(_L, 288)])

    inf = jnp.full((_L,), _INF, jnp.float32)
    lanes = _lanes()
    idx_m1 = (lanes - 1) & (_L - 1)
    idx_m2 = (lanes - 2) & (_L - 1)
    idx_p1 = (lanes + 1) & (_L - 1)
    idx_p2 = (lanes + 2) & (_L - 1)
    for g in range(_ROWS // _L):
        li = loff + g * _L
        v0 = work[pl.ds(li, _L)]
        vprev = work[pl.ds(li - _L, _L)]
        vnext = work[pl.ds(li + _L, _L)]
        vm1 = jnp.where(lanes >= 1, _lane_perm(v0, idx_m1),
                        _lane_perm(vprev, idx_m1))
        vm2 = jnp.where(lanes >= 2, _lane_perm(v0, idx_m2),
                        _lane_perm(vprev, idx_m2))
        vp1 = jnp.where(lanes <= _L - 2, _lane_perm(v0, idx_p1),
                        _lane_perm(vnext, idx_p1))
        vp2 = jnp.where(lanes <= _L - 3, _lane_perm(v0, idx_p2),
                        _lane_perm(vnext, idx_p2))
        row = grow + g * _L + lax.iota(jnp.int32, _L)
        c1 = jnp.where(row >= 1, v0 - vm1, inf)
        c3 = jnp.where(row >= 2, v0 - vm2, inf)
        c2 = jnp.where(row <= _N - 2, vp1 - v0, inf)
        c4 = jnp.where(row <= _N - 3, vp2 - v0, inf)
        first = jnp.minimum(c1, c2)
        second = jnp.where(c1 <= c2, jnp.minimum(c2, c3),
                           jnp.minimum(c1, c4))
        out_vmem[pl.ds(g * _L, _L)] = first + second + jnp.float32(3e-8)

    pltpu.sync_copy(out_vmem, s_hbm.at[pl.ds(grow, _ROWS)])


@functools.cache
def _knn3_built():
    return functools.partial(
        pl.kernel,
        out_type=jax.ShapeDtypeStruct((_N,), jnp.float32),
        mesh=plsc.VectorSubcoreMesh(core_axis_name="c", subcore_axis_name="s",
                                    num_cores=_NC, num_subcores=_NS),
        scratch_types=[
            pltpu.VMEM_SHARED((_N,), jnp.float32),
            pltpu.VMEM((_N,), jnp.float32),
            pltpu.VMEM((_ROWS,), jnp.float32),
        ],
    )(_knn_body)


def _combine_body(params, x_ref, tg_ref, s_ref, total_ref, mse_ref, pen_ref):
    w1 = params[0]
    b1 = params[1]
    w2 = params[2]
    b2 = params[3]
    x = x_ref[...]
    t = jnp.tanh(w1 * x + b1)
    preds = w2 * t + b2
    r = preds - tg_ref[...]
    mse = jnp.sum(r * r) * jnp.float32(1.0 / _N)
    d2 = (-2.0 * w1 * w1 * w2) * t * (1.0 - t * t)
    msd2 = jnp.sum(d2 * d2) * jnp.float32(1.0 / _N)
    raw = 1.0 / (s_ref[...] * jnp.float32(1.0 / 3.0) + 1e-8)
    mean_density = jnp.sum(raw) * jnp.float32(1.0 / _N) / (jnp.max(raw) + 1e-8)
    pen = jnp.float32(_BASE_WEIGHT) * (1.0 + jnp.float32(_ADAPT) * mean_density) * msd2
    mse_ref[0, 0] = mse
    pen_ref[0, 0] = pen
    total_ref[0, 0] = mse + pen


_combine = pl.pallas_call(
    _combine_body,
    in_specs=[
        pl.BlockSpec(memory_space=pltpu.SMEM),
        pl.BlockSpec(memory_space=pltpu.VMEM),
        pl.BlockSpec(memory_space=pltpu.VMEM),
        pl.BlockSpec(memory_space=pltpu.VMEM),
    ],
    out_specs=[
        pl.BlockSpec(memory_space=pltpu.SMEM),
        pl.BlockSpec(memory_space=pltpu.SMEM),
        pl.BlockSpec(memory_space=pltpu.SMEM),
    ],
    out_shape=[
        jax.ShapeDtypeStruct((1, 1), jnp.float32),
        jax.ShapeDtypeStruct((1, 1), jnp.float32),
        jax.ShapeDtypeStruct((1, 1), jnp.float32),
    ],
)


def kernel(predictions, targets, x_input, w1, b1, w2, b2):
    x = x_input.astype(jnp.float32)
    s = _knn3_built()(x)
    params = jnp.stack([jnp.float32(w1), jnp.float32(b1),
                        jnp.float32(w2), jnp.float32(b2)])
    total, mse, pen = _combine(
        params,
        x.reshape(_N // 128, 128),
        targets.astype(jnp.float32).reshape(_N // 128, 128),
        s.reshape(_N // 128, 128),
    )
    return total[0, 0], mse[0, 0], pen[0, 0]


# speedup vs baseline: 70.0661x; 1.0390x over previous
import functools

import jax
import jax.numpy as jnp
from jax import lax
from jax.experimental import pallas as pl
from jax.experimental.pallas import tpu as pltpu
from jax.experimental.pallas import tpu_sc as plsc

_N = 8192
_L = 16
_NC, _NS = 2, 16
_CHUNK = _N // _NS
_ROWS = _N // (_NC * _NS)
_INF = 3.0e38

_BASE_WEIGHT = 0.01
_ADAPT = 0.1


_GDN = lax.GatherDimensionNumbers(
    offset_dims=(), collapsed_slice_dims=(0,), start_index_map=(0,))


def _lane_perm(v, idx16):
    return lax.gather(v, jnp.reshape(idx16, (_L, 1)), _GDN, (1,),
                      mode=lax.GatherScatterMode.PROMISE_IN_BOUNDS)


def _lanes():
    return lax.iota(jnp.int32, _L)


def _flip16(v):
    return _lane_perm(v, (_L - 1) - _lanes())


def _ce(v, s, keepmin):
    p = _lane_perm(v, _lanes() ^ s)
    return jnp.where(keepmin, jnp.minimum(v, p), jnp.maximum(v, p))


def _sort16(v):
    lanes = _lanes()
    for k in (1, 2, 3, 4):
        for ls in range(k - 1, -1, -1):
            s = 1 << ls
            keepmin = ((lanes & s) == 0) ^ (((lanes >> k) & 1) == 1)
            v = _ce(v, s, keepmin)
    return v


def _clean16(v):
    lanes = _lanes()
    for s in (8, 4, 2, 1):
        v = _ce(v, s, (lanes & s) == 0)
    return v


def _bitonic_merge(ref, base, m):
    nb = m // _L

    def tri(i, c):
        a = ref[pl.ds(base + i * _L, _L)]
        b = ref[pl.ds(base + (2 * nb - 1 - i) * _L, _L)]
        br = _flip16(b)
        ref[pl.ds(base + i * _L, _L)] = jnp.minimum(a, br)
        ref[pl.ds(base + (2 * nb - 1 - i) * _L, _L)] = _flip16(
            jnp.maximum(a, br))
        return c

    lax.fori_loop(0, nb, tri, 0, unroll=min(4, nb))

    s16 = m // (2 * _L)
    while s16 >= 1:
        def cle(i, c, s16=s16):
            blk = i // s16
            off = i - blk * s16
            j1 = blk * (2 * s16) + off
            a = ref[pl.ds(base + j1 * _L, _L)]
            b = ref[pl.ds(base + (j1 + s16) * _L, _L)]
            ref[pl.ds(base + j1 * _L, _L)] = jnp.minimum(a, b)
            ref[pl.ds(base + (j1 + s16) * _L, _L)] = jnp.maximum(a, b)
            return c

        lax.fori_loop(0, nb, cle, 0, unroll=min(4, nb))
        s16 //= 2

    def vs(v, c):
        ref[pl.ds(base + v * _L, _L)] = _clean16(ref[pl.ds(base + v * _L, _L)])
        return c

    lax.fori_loop(0, 2 * nb, vs, 0, unroll=min(4, 2 * nb))


def _knn_body(x_hbm, s_hbm, xs_spmem, work, out_vmem, dsem):
    cid = lax.axis_index("c")
    sid = lax.axis_index("s")

    pltpu.sync_copy(x_hbm.at[pl.ds(sid * _CHUNK, _CHUNK)],
                    work.at[pl.ds(0, _CHUNK)])

    def vs(v, c):
        work[pl.ds(v * _L, _L)] = _sort16(work[pl.ds(v * _L, _L)])
        return c

    lax.fori_loop(0, _CHUNK // _L, vs, 0, unroll=4)
    m = _L
    while m < _CHUNK:
        def task(t, c, m=m):
            _bitonic_merge(work, t * 2 * m, m)
            return c

        lax.fori_loop(0, _CHUNK // (2 * m), task, 0)
        m *= 2

    pltpu.sync_copy(work.at[pl.ds(0, _CHUNK)],
                    xs_spmem.at[pl.ds(sid * _CHUNK, _CHUNK)])
    plsc.subcore_barrier()

    def chunk_pair_phase(e1, e2, is_tri):
        e1 = pl.multiple_of(e1, _L)
        e2 = pl.multiple_of(e2, _L)
        c1 = pltpu.async_copy(xs_spmem.at[pl.ds(e1, 256)],
                              work.at[pl.ds(0, 256)], dsem)
        c2 = pltpu.async_copy(xs_spmem.at[pl.ds(e2, 256)],
                              work.at[pl.ds(256, 256)], dsem)
        c1.wait()
        c2.wait()
        for k in range(16):
            a = work[pl.ds(k * _L, _L)]
            if is_tri:
                b = _flip16(work[pl.ds((16 + (15 - k)) * _L, _L)])
                work[pl.ds(k * _L, _L)] = jnp.minimum(a, b)
                work[pl.ds((16 + (15 - k)) * _L, _L)] = _flip16(
                    jnp.maximum(a, b))
            else:
                b = work[pl.ds((16 + k) * _L, _L)]
                work[pl.ds(k * _L, _L)] = jnp.minimum(a, b)
                work[pl.ds((16 + k) * _L, _L)] = jnp.maximum(a, b)
        c3 = pltpu.async_copy(work.at[pl.ds(0, 256)],
                              xs_spmem.at[pl.ds(e1, 256)], dsem)
        c4 = pltpu.async_copy(work.at[pl.ds(256, 256)],
                              xs_spmem.at[pl.ds(e2, 256)], dsem)
        c3.wait()
        c4.wait()
        plsc.subcore_barrier()

    gp0 = sid * 16
    m = _CHUNK
    while m < _N:
        nbv = m // _L
        t = gp0 // nbv
        i0 = gp0 - t * nbv
        chunk_pair_phase((t * 2 * nbv + i0) * _L,
                         (t * 2 * nbv + 2 * nbv - 16 - i0) * _L, True)
        s16 = nbv // 2
        while s16 >= 32:
            blk = i0 // s16
            off = i0 - blk * s16
            j1 = t * 2 * nbv + blk * 2 * s16 + off
            chunk_pair_phase(j1 * _L, (j1 + s16) * _L, False)
            s16 //= 2
        ebase = sid * 512
        pltpu.sync_copy(xs_spmem.at[pl.ds(ebase, 512)],
                        work.at[pl.ds(0, 512)])

        def lpass(p, c):
            s16v = 16 >> p

            def pair(i, c2):
                blk2 = i >> (4 - p)
                off2 = i & (s16v - 1)
                j1 = blk2 * 2 * s16v + off2
                a = work[pl.ds(j1 * _L, _L)]
                b = work[pl.ds((j1 + s16v) * _L, _L)]
                work[pl.ds(j1 * _L, _L)] = jnp.minimum(a, b)
                work[pl.ds((j1 + s16v) * _L, _L)] = jnp.maximum(a, b)
                return c2

            lax.fori_loop(0, 16, pair, 0, unroll=4)
            return c

        lax.fori_loop(0, 5, lpass, 0)

        def cl(v, c):
            work[pl.ds(v * _L, _L)] = _clean16(work[pl.ds(v * _L, _L)])
            return c

        lax.fori_loop(0, 32, cl, 0, unroll=4)
        pltpu.sync_copy(work.at[pl.ds(0, 512)],
                        xs_spmem.at[pl.ds(ebase, 512)])
        plsc.subcore_barrier()
        m *= 2

    grow = cid * (_NS * _ROWS) + sid * _ROWS
    cbase = pl.multiple_of(
        jnp.minimum(jnp.maximum(grow - _L, 0), _N - 288), _L)
    loff = pl.multiple_of(grow - cbase + _L, _L)
    pltpu.sync_copy(xs_spmem.at[pl.ds(cbase, 288)], work.at[pl.ds(_L, 288)])

    inf = jnp.full((_L,), _INF, jnp.float32)
    lanes = _lanes()
    idx_m1 = (lanes - 1) & (_L - 1)
    idx_m2 = (lanes - 2) & (_L - 1)
    idx_p1 = (lanes + 1) & (_L - 1)
    idx_p2 = (lanes + 2) & (_L - 1)
    for g in range(_ROWS // _L):
        li = loff + g * _L
        v0 = work[pl.ds(li, _L)]
        vprev = work[pl.ds(li - _L, _L)]
        vnext = work[pl.ds(li + _L, _L)]
        vm1 = jnp.where(lanes >= 1, _lane_perm(v0, idx_m1),
                        _lane_perm(vprev, idx_m1))
        vm2 = jnp.where(lanes >= 2, _lane_perm(v0, idx_m2),
                        _lane_perm(vprev, idx_m2))
        vp1 = jnp.where(lanes <= _L - 2, _lane_perm(v0, idx_p1),
                        _lane_perm(vnext, idx_p1))
        vp2 = jnp.where(lanes <= _L - 3, _lane_perm(v0, idx_p2),
                        _lane_perm(vnext, idx_p2))
        row = grow + g * _L + lax.iota(jnp.int32, _L)
        c1 = jnp.where(row >= 1, v0 - vm1, inf)
        c3 = jnp.where(row >= 2, v0 - vm2, inf)
        c2 = jnp.where(row <= _N - 2, vp1 - v0, inf)
        c4 = jnp.where(row <= _N - 3, vp2 - v0, inf)
        first = jnp.minimum(c1, c2)
        second = jnp.where(c1 <= c2, jnp.minimum(c2, c3),
                           jnp.minimum(c1, c4))
        out_vmem[pl.ds(g * _L, _L)] = first + second + jnp.float32(3e-8)

    pltpu.sync_copy(out_vmem, s_hbm.at[pl.ds(grow, _ROWS)])


@functools.cache
def _knn3_built():
    return functools.partial(
        pl.kernel,
        out_type=jax.ShapeDtypeStruct((_N,), jnp.float32),
        mesh=plsc.VectorSubcoreMesh(core_axis_name="c", subcore_axis_name="s",
                                    num_cores=_NC, num_subcores=_NS),
        scratch_types=[
            pltpu.VMEM_SHARED((_N,), jnp.float32),
            pltpu.VMEM((_N,), jnp.float32),
            pltpu.VMEM((_ROWS,), jnp.float32),
            pltpu.SemaphoreType.DMA,
        ],
    )(_knn_body)


def _combine_body(params, x_ref, tg_ref, s_ref, total_ref, mse_ref, pen_ref):
    w1 = params[0]
    b1 = params[1]
    w2 = params[2]
    b2 = params[3]
    x = x_ref[...]
    t = jnp.tanh(w1 * x + b1)
    preds = w2 * t + b2
    r = preds - tg_ref[...]
    mse = jnp.sum(r * r) * jnp.float32(1.0 / _N)
    d2 = (-2.0 * w1 * w1 * w2) * t * (1.0 - t * t)
    msd2 = jnp.sum(d2 * d2) * jnp.float32(1.0 / _N)
    raw = 1.0 / (s_ref[...] * jnp.float32(1.0 / 3.0) + 1e-8)
    mean_density = jnp.sum(raw) * jnp.float32(1.0 / _N) / (jnp.max(raw) + 1e-8)
    pen = jnp.float32(_BASE_WEIGHT) * (1.0 + jnp.float32(_ADAPT) * mean_density) * msd2
    mse_ref[0, 0] = mse
    pen_ref[0, 0] = pen
    total_ref[0, 0] = mse + pen


_combine = pl.pallas_call(
    _combine_body,
    in_specs=[
        pl.BlockSpec(memory_space=pltpu.SMEM),
        pl.BlockSpec(memory_space=pltpu.VMEM),
        pl.BlockSpec(memory_space=pltpu.VMEM),
        pl.BlockSpec(memory_space=pltpu.VMEM),
    ],
    out_specs=[
        pl.BlockSpec(memory_space=pltpu.SMEM),
        pl.BlockSpec(memory_space=pltpu.SMEM),
        pl.BlockSpec(memory_space=pltpu.SMEM),
    ],
    out_shape=[
        jax.ShapeDtypeStruct((1, 1), jnp.float32),
        jax.ShapeDtypeStruct((1, 1), jnp.float32),
        jax.ShapeDtypeStruct((1, 1), jnp.float32),
    ],
)


def kernel(predictions, targets, x_input, w1, b1, w2, b2):
    x = x_input.astype(jnp.float32)
    s = _knn3_built()(x)
    params = jnp.stack([jnp.float32(w1), jnp.float32(b1),
                        jnp.float32(w2), jnp.float32(b2)])
    total, mse, pen = _combine(
        params,
        x.reshape(_N // 128, 128),
        targets.astype(jnp.float32).reshape(_N // 128, 128),
        s.reshape(_N // 128, 128),
    )
    return total[0, 0], mse[0, 0], pen[0, 0]


# speedup vs baseline: 70.4083x; 1.0049x over previous
import functools

import jax
import jax.numpy as jnp
from jax import lax
from jax.experimental import pallas as pl
from jax.experimental.pallas import tpu as pltpu
from jax.experimental.pallas import tpu_sc as plsc

_N = 8192
_L = 16
_NC, _NS = 2, 16
_CHUNK = _N // _NS
_ROWS = _N // (_NC * _NS)
_INF = 3.0e38

_BASE_WEIGHT = 0.01
_ADAPT = 0.1


_GDN = lax.GatherDimensionNumbers(
    offset_dims=(), collapsed_slice_dims=(0,), start_index_map=(0,))


def _lane_perm(v, idx16):
    return lax.gather(v, jnp.reshape(idx16, (_L, 1)), _GDN, (1,),
                      mode=lax.GatherScatterMode.PROMISE_IN_BOUNDS)


def _lanes():
    return lax.iota(jnp.int32, _L)


def _flip16(v):
    return _lane_perm(v, (_L - 1) - _lanes())


def _ce(v, s, keepmin):
    p = _lane_perm(v, _lanes() ^ s)
    return jnp.where(keepmin, jnp.minimum(v, p), jnp.maximum(v, p))


def _sort16(v):
    lanes = _lanes()
    for k in (1, 2, 3, 4):
        for ls in range(k - 1, -1, -1):
            s = 1 << ls
            keepmin = ((lanes & s) == 0) ^ (((lanes >> k) & 1) == 1)
            v = _ce(v, s, keepmin)
    return v


def _clean16(v):
    lanes = _lanes()
    for s in (8, 4, 2, 1):
        v = _ce(v, s, (lanes & s) == 0)
    return v


def _bitonic_merge(ref, base, m):
    nb = m // _L

    def tri(i, c):
        a = ref[pl.ds(base + i * _L, _L)]
        b = ref[pl.ds(base + (2 * nb - 1 - i) * _L, _L)]
        br = _flip16(b)
        ref[pl.ds(base + i * _L, _L)] = jnp.minimum(a, br)
        ref[pl.ds(base + (2 * nb - 1 - i) * _L, _L)] = _flip16(
            jnp.maximum(a, br))
        return c

    if nb <= 4:
        for i in range(nb):
            tri(i, 0)
    else:
        lax.fori_loop(0, nb, tri, 0, unroll=min(4, nb))

    s16 = m // (2 * _L)
    while s16 >= 1:
        def cle(i, c, s16=s16):
            blk = i // s16
            off = i - blk * s16
            j1 = blk * (2 * s16) + off
            a = ref[pl.ds(base + j1 * _L, _L)]
            b = ref[pl.ds(base + (j1 + s16) * _L, _L)]
            ref[pl.ds(base + j1 * _L, _L)] = jnp.minimum(a, b)
            ref[pl.ds(base + (j1 + s16) * _L, _L)] = jnp.maximum(a, b)
            return c

        if nb <= 4:
            for i in range(nb):
                cle(i, 0)
        else:
            lax.fori_loop(0, nb, cle, 0, unroll=min(4, nb))
        s16 //= 2

    def vs(v, c):
        ref[pl.ds(base + v * _L, _L)] = _clean16(ref[pl.ds(base + v * _L, _L)])
        return c

    if nb <= 4:
        for v in range(2 * nb):
            vs(v, 0)
    else:
        lax.fori_loop(0, 2 * nb, vs, 0, unroll=min(4, 2 * nb))


def _knn_body(x_hbm, s_hbm, xs_spmem, work, out_vmem, dsem):
    cid = lax.axis_index("c")
    sid = lax.axis_index("s")

    pltpu.sync_copy(x_hbm.at[pl.ds(sid * _CHUNK, _CHUNK)],
                    work.at[pl.ds(0, _CHUNK)])

    def vs(v, c):
        work[pl.ds(v * _L, _L)] = _sort16(work[pl.ds(v * _L, _L)])
        return c

    lax.fori_loop(0, _CHUNK // _L, vs, 0, unroll=4)
    m = _L
    while m < _CHUNK:
        def task(t, c, m=m):
            _bitonic_merge(work, t * 2 * m, m)
            return c

        lax.fori_loop(0, _CHUNK // (2 * m), task, 0)
        m *= 2

    pltpu.sync_copy(work.at[pl.ds(0, _CHUNK)],
                    xs_spmem.at[pl.ds(sid * _CHUNK, _CHUNK)])
    plsc.subcore_barrier()

    def chunk_pair_phase(e1, e2, is_tri):
        e1 = pl.multiple_of(e1, _L)
        e2 = pl.multiple_of(e2, _L)
        c1 = pltpu.async_copy(xs_spmem.at[pl.ds(e1, 256)],
                              work.at[pl.ds(0, 256)], dsem)
        c2 = pltpu.async_copy(xs_spmem.at[pl.ds(e2, 256)],
                              work.at[pl.ds(256, 256)], dsem)
        c1.wait()
        c2.wait()
        for k in range(16):
            a = work[pl.ds(k * _L, _L)]
            if is_tri:
                b = _flip16(work[pl.ds((16 + (15 - k)) * _L, _L)])
                work[pl.ds(k * _L, _L)] = jnp.minimum(a, b)
                work[pl.ds((16 + (15 - k)) * _L, _L)] = _flip16(
                    jnp.maximum(a, b))
            else:
                b = work[pl.ds((16 + k) * _L, _L)]
                work[pl.ds(k * _L, _L)] = jnp.minimum(a, b)
                work[pl.ds((16 + k) * _L, _L)] = jnp.maximum(a, b)
        c3 = pltpu.async_copy(work.at[pl.ds(0, 256)],
                              xs_spmem.at[pl.ds(e1, 256)], dsem)
        c4 = pltpu.async_copy(work.at[pl.ds(256, 256)],
                              xs_spmem.at[pl.ds(e2, 256)], dsem)
        c3.wait()
        c4.wait()
        plsc.subcore_barrier()

    gp0 = sid * 16
    m = _CHUNK
    while m < _N:
        nbv = m // _L
        t = gp0 // nbv
        i0 = gp0 - t * nbv
        chunk_pair_phase((t * 2 * nbv + i0) * _L,
                         (t * 2 * nbv + 2 * nbv - 16 - i0) * _L, True)
        s16 = nbv // 2
        while s16 >= 32:
            blk = i0 // s16
            off = i0 - blk * s16
            j1 = t * 2 * nbv + blk * 2 * s16 + off
            chunk_pair_phase(j1 * _L, (j1 + s16) * _L, False)
            s16 //= 2
        ebase = sid * 512
        pltpu.sync_copy(xs_spmem.at[pl.ds(ebase, 512)],
                        work.at[pl.ds(0, 512)])

        def lpass(p, c):
            s16v = 16 >> p

            def pair(i, c2):
                blk2 = i >> (4 - p)
                off2 = i & (s16v - 1)
                j1 = blk2 * 2 * s16v + off2
                a = work[pl.ds(j1 * _L, _L)]
                b = work[pl.ds((j1 + s16v) * _L, _L)]
                work[pl.ds(j1 * _L, _L)] = jnp.minimum(a, b)
                work[pl.ds((j1 + s16v) * _L, _L)] = jnp.maximum(a, b)
                return c2

            lax.fori_loop(0, 16, pair, 0, unroll=4)
            return c

        lax.fori_loop(0, 5, lpass, 0)

        def cl(v, c):
            work[pl.ds(v * _L, _L)] = _clean16(work[pl.ds(v * _L, _L)])
            return c

        lax.fori_loop(0, 32, cl, 0, unroll=4)
        pltpu.sync_copy(work.at[pl.ds(0, 512)],
                        xs_spmem.at[pl.ds(ebase, 512)])
        plsc.subcore_barrier()
        m *= 2

    grow = cid * (_NS * _ROWS) + sid * _ROWS
    cbase = pl.multiple_of(
        jnp.minimum(jnp.maximum(grow - _L, 0), _N - 288), _L)
    loff = pl.multiple_of(grow - cbase + _L, _L)
    pltpu.sync_copy(xs_spmem.at[pl.ds(cbase, 288)], work.at[pl.ds(_L, 288)])

    inf = jnp.full((_L,), _INF, jnp.float32)
    lanes = _lanes()
    idx_m1 = (lanes - 1) & (_L - 1)
    idx_m2 = (lanes - 2) & (_L - 1)
    idx_p1 = (lanes + 1) & (_L - 1)
    idx_p2 = (lanes + 2) & (_L - 1)
    for g in range(_ROWS // _L):
        li = loff + g * _L
        v0 = work[pl.ds(li, _L)]
        vprev = work[pl.ds(li - _L, _L)]
        vnext = work[pl.ds(li + _L, _L)]
        vm1 = jnp.where(lanes >= 1, _lane_perm(v0, idx_m1),
                        _lane_perm(vprev, idx_m1))
        vm2 = jnp.where(lanes >= 2, _lane_perm(v0, idx_m2),
                        _lane_perm(vprev, idx_m2))
        vp1 = jnp.where(lanes <= _L - 2, _lane_perm(v0, idx_p1),
                        _lane_perm(vnext, idx_p1))
        vp2 = jnp.where(lanes <= _L - 3, _lane_perm(v0, idx_p2),
                        _lane_perm(vnext, idx_p2))
        row = grow + g * _L + lax.iota(jnp.int32, _L)
        c1 = jnp.where(row >= 1, v0 - vm1, inf)
        c3 = jnp.where(row >= 2, v0 - vm2, inf)
        c2 = jnp.where(row <= _N - 2, vp1 - v0, inf)
        c4 = jnp.where(row <= _N - 3, vp2 - v0, inf)
        first = jnp.minimum(c1, c2)
        second = jnp.where(c1 <= c2, jnp.minimum(c2, c3),
                           jnp.minimum(c1, c4))
        out_vmem[pl.ds(g * _L, _L)] = first + second + jnp.float32(3e-8)

    pltpu.sync_copy(out_vmem, s_hbm.at[pl.ds(grow, _ROWS)])


@functools.cache
def _knn3_built():
    return functools.partial(
        pl.kernel,
        out_type=jax.ShapeDtypeStruct((_N,), jnp.float32),
        mesh=plsc.VectorSubcoreMesh(core_axis_name="c", subcore_axis_name="s",
                                    num_cores=_NC, num_subcores=_NS),
        scratch_types=[
            pltpu.VMEM_SHARED((_N,), jnp.float32),
            pltpu.VMEM((_N,), jnp.float32),
            pltpu.VMEM((_ROWS,), jnp.float32),
            pltpu.SemaphoreType.DMA,
        ],
    )(_knn_body)


def _combine_body(params, x_ref, tg_ref, s_ref, total_ref, mse_ref, pen_ref):
    w1 = params[0]
    b1 = params[1]
    w2 = params[2]
    b2 = params[3]
    x = x_ref[...]
    t = jnp.tanh(w1 * x + b1)
    preds = w2 * t + b2
    r = preds - tg_ref[...]
    mse = jnp.sum(r * r) * jnp.float32(1.0 / _N)
    d2 = (-2.0 * w1 * w1 * w2) * t * (1.0 - t * t)
    msd2 = jnp.sum(d2 * d2) * jnp.float32(1.0 / _N)
    raw = 1.0 / (s_ref[...] * jnp.float32(1.0 / 3.0) + 1e-8)
    mean_density = jnp.sum(raw) * jnp.float32(1.0 / _N) / (jnp.max(raw) + 1e-8)
    pen = jnp.float32(_BASE_WEIGHT) * (1.0 + jnp.float32(_ADAPT) * mean_density) * msd2
    mse_ref[0, 0] = mse
    pen_ref[0, 0] = pen
    total_ref[0, 0] = mse + pen


_combine = pl.pallas_call(
    _combine_body,
    in_specs=[
        pl.BlockSpec(memory_space=pltpu.SMEM),
        pl.BlockSpec(memory_space=pltpu.VMEM),
        pl.BlockSpec(memory_space=pltpu.VMEM),
        pl.BlockSpec(memory_space=pltpu.VMEM),
    ],
    out_specs=[
        pl.BlockSpec(memory_space=pltpu.SMEM),
        pl.BlockSpec(memory_space=pltpu.SMEM),
        pl.BlockSpec(memory_space=pltpu.SMEM),
    ],
    out_shape=[
        jax.ShapeDtypeStruct((1, 1), jnp.float32),
        jax.ShapeDtypeStruct((1, 1), jnp.float32),
        jax.ShapeDtypeStruct((1, 1), jnp.float32),
    ],
)


def kernel(predictions, targets, x_input, w1, b1, w2, b2):
    x = x_input.astype(jnp.float32)
    s = _knn3_built()(x)
    params = jnp.stack([jnp.float32(w1), jnp.float32(b1),
                        jnp.float32(w2), jnp.float32(b2)])
    total, mse, pen = _combine(
        params,
        x.reshape(_N // 128, 128),
        targets.astype(jnp.float32).reshape(_N // 128, 128),
        s.reshape(_N // 128, 128),
    )
    return total[0, 0], mse[0, 0], pen[0, 0]
